# Initial kernel scaffold; baseline (speedup 1.0000x reference)
import jax
import jax.numpy as jnp
from jax.experimental import pallas as pl


def kernel(x, edge_index, W1, b1, W2, b2, Wf, bf):
    raise NotImplementedError("write your pallas kernel here")


# speedup vs baseline: 5.0723x; 5.0723x over previous
import functools

import jax
import jax.numpy as jnp
from jax import lax
from jax.experimental import pallas as pl
from jax.experimental.pallas import tpu as pltpu
from jax.experimental.pallas import tpu_sc as plsc

N = 10000
E = 320000
D = 128
H = 128
C = 10

NC = 2
NS = 16
NW = NC * NS

EDGES_PER_W = E // NW
CHUNK = 80
CHUNKS_PER_W = EDGES_PER_W // CHUNK
ROWS_PER_TILE = 632
N_PAD = ROWS_PER_TILE * NS


def _sc_segment_sum_body(h_hbm, src_hbm, dst_hbm, zero_hbm, out_hbm,
                         src_v, dst_v, rows_v, agg_sh, sem):
    cid = lax.axis_index("c")
    sid = lax.axis_index("s")
    wid = cid * NS + sid

    pltpu.sync_copy(zero_hbm.at[pl.ds(sid * ROWS_PER_TILE, ROWS_PER_TILE)],
                    agg_sh.at[pl.ds(sid * ROWS_PER_TILE, ROWS_PER_TILE)])
    plsc.subcore_barrier()

    edge_base = wid * EDGES_PER_W

    def body(i, _):
        base = edge_base + i * CHUNK
        pltpu.sync_copy(src_hbm.at[pl.ds(base, CHUNK)], src_v)
        pltpu.sync_copy(dst_hbm.at[pl.ds(base, CHUNK)], dst_v)
        pltpu.async_copy(h_hbm.at[src_v], rows_v, sem).wait()
        pltpu.sync_copy(rows_v, agg_sh.at[dst_v], add=True)
        return 0

    lax.fori_loop(0, CHUNKS_PER_W, body, 0)
    plsc.subcore_barrier()

    row0 = sid * ROWS_PER_TILE
    pltpu.sync_copy(agg_sh.at[pl.ds(row0, ROWS_PER_TILE)],
                    out_hbm.at[pl.ds(cid * N_PAD + row0, ROWS_PER_TILE)])


_sc_segment_sum = functools.partial(
    pl.kernel,
    out_type=jax.ShapeDtypeStruct((NC * N_PAD, D), jnp.float32),
    mesh=plsc.VectorSubcoreMesh(core_axis_name="c", subcore_axis_name="s"),
    scratch_types=[
        pltpu.VMEM((CHUNK,), jnp.int32),
        pltpu.VMEM((CHUNK,), jnp.int32),
        pltpu.VMEM((CHUNK, D), jnp.float32),
        pltpu.VMEM_SHARED((N_PAD, D), jnp.float32),
        pltpu.SemaphoreType.DMA,
    ],
)(_sc_segment_sum_body)


BN = 2000


def _mm_relu_body(h_ref, p0_ref, p1_ref, w_ref, b_ref, o_ref):
    s = h_ref[...] + p0_ref[...] + p1_ref[...]
    y = jnp.dot(s, w_ref[...], preferred_element_type=jnp.float32) + b_ref[...]
    o_ref[...] = jnp.maximum(y, 0.0)


def _mm_relu(h, p0, p1, w, b):
    return pl.pallas_call(
        _mm_relu_body,
        out_shape=jax.ShapeDtypeStruct((N, H), jnp.float32),
        grid=(N // BN,),
        in_specs=[
            pl.BlockSpec((BN, D), lambda i: (i, 0)),
            pl.BlockSpec((BN, D), lambda i: (i, 0)),
            pl.BlockSpec((BN, D), lambda i: (i, 0)),
            pl.BlockSpec((D, H), lambda i: (0, 0)),
            pl.BlockSpec((1, H), lambda i: (0, 0)),
        ],
        out_specs=pl.BlockSpec((BN, H), lambda i: (i, 0)),
    )(h, p0, p1, w, b.reshape(1, H))


def _mm2_head_body(h_ref, p0_ref, p1_ref, w_ref, b_ref, wf_ref, bf_ref,
                   o_ref, acc_ref):
    i = pl.program_id(0)
    s = h_ref[...] + p0_ref[...] + p1_ref[...]
    y = jnp.dot(s, w_ref[...], preferred_element_type=jnp.float32) + b_ref[...]
    h2 = jnp.maximum(y, 0.0)
    colsum = jnp.sum(h2, axis=0, keepdims=True)

    @pl.when(i == 0)
    def _():
        acc_ref[...] = colsum

    @pl.when(i > 0)
    def _():
        acc_ref[...] = acc_ref[...] + colsum

    @pl.when(i == pl.num_programs(0) - 1)
    def _():
        mean = acc_ref[...] * (1.0 / N)
        o_ref[...] = (jnp.dot(mean, wf_ref[...],
                              preferred_element_type=jnp.float32) + bf_ref[...])


def _mm2_head(h, p0, p1, w, b, wf, bf):
    return pl.pallas_call(
        _mm2_head_body,
        out_shape=jax.ShapeDtypeStruct((1, C), jnp.float32),
        grid=(N // BN,),
        in_specs=[
            pl.BlockSpec((BN, D), lambda i: (i, 0)),
            pl.BlockSpec((BN, D), lambda i: (i, 0)),
            pl.BlockSpec((BN, D), lambda i: (i, 0)),
            pl.BlockSpec((D, H), lambda i: (0, 0)),
            pl.BlockSpec((1, H), lambda i: (0, 0)),
            pl.BlockSpec((H, C), lambda i: (0, 0)),
            pl.BlockSpec((1, C), lambda i: (0, 0)),
        ],
        out_specs=pl.BlockSpec((1, C), lambda i: (0, 0)),
        scratch_shapes=[pltpu.VMEM((1, H), jnp.float32)],
    )(h, p0, p1, w, b.reshape(1, H), wf, bf.reshape(1, C))


def kernel(x, edge_index, W1, b1, W2, b2, Wf, bf):
    src = edge_index[0].astype(jnp.int32)
    dst = edge_index[1].astype(jnp.int32)
    zeros = jnp.zeros((N_PAD, D), jnp.float32)

    p = _sc_segment_sum(x, src, dst, zeros)
    h1 = _mm_relu(x, p[:N], p[N_PAD:N_PAD + N], W1, b1)
    p2 = _sc_segment_sum(h1, src, dst, zeros)
    return _mm2_head(h1, p2[:N], p2[N_PAD:N_PAD + N], W2, b2, Wf, bf)


# speedup vs baseline: 8.8468x; 1.7441x over previous
import functools

import jax
import jax.numpy as jnp
from jax import lax
from jax.experimental import pallas as pl
from jax.experimental.pallas import tpu as pltpu
from jax.experimental.pallas import tpu_sc as plsc

N = 10000
E = 320000
D = 128
H = 128
C = 10

NC = 2
NS = 16
NW = NC * NS

EDGES_PER_W = E // NW
CHUNK = 80
CHUNKS_PER_W = EDGES_PER_W // CHUNK
SB_N = 5
SB_C = CHUNKS_PER_W // SB_N
ROWS_PER_TILE = 632
N_PAD = ROWS_PER_TILE * NS


def _sc_segment_sum_body(h_hbm, src_hbm, dst_hbm, zero_hbm, out_hbm,
                         src_v, dst_v, rows_a, rows_b, agg_sh, sem_a, sem_b):
    cid = lax.axis_index("c")
    sid = lax.axis_index("s")
    wid = cid * NS + sid

    pltpu.sync_copy(zero_hbm.at[pl.ds(sid * ROWS_PER_TILE, ROWS_PER_TILE)],
                    agg_sh.at[pl.ds(sid * ROWS_PER_TILE, ROWS_PER_TILE)])
    plsc.subcore_barrier()

    def _gather(i, buf, sem):
        return pltpu.async_copy(h_hbm.at[src_v.at[i]], buf, sem)

    def _wait(buf, sem):
        pltpu.make_async_copy(h_hbm.at[pl.ds(0, CHUNK)], buf, sem).wait()

    def _scatter(i, buf):
        pltpu.sync_copy(buf, agg_sh.at[dst_v.at[i]], add=True)

    def sb_body(sb, _):
        pltpu.sync_copy(src_hbm.at[wid, sb], src_v)
        pltpu.sync_copy(dst_hbm.at[wid, sb], dst_v)
        _gather(0, rows_a, sem_a)

        def body(k, _):
            i0 = 2 * k
            _wait(rows_a, sem_a)
            _gather(i0 + 1, rows_b, sem_b)
            _scatter(i0, rows_a)
            _wait(rows_b, sem_b)
            _gather(i0 + 2, rows_a, sem_a)
            _scatter(i0 + 1, rows_b)
            return 0

        lax.fori_loop(0, (SB_C - 1) // 2, body, 0)
        _wait(rows_a, sem_a)
        _scatter(SB_C - 1, rows_a)
        return 0

    lax.fori_loop(0, SB_N, sb_body, 0)
    plsc.subcore_barrier()

    row0 = sid * ROWS_PER_TILE
    pltpu.sync_copy(agg_sh.at[pl.ds(row0, ROWS_PER_TILE)],
                    out_hbm.at[pl.ds(cid * N_PAD + row0, ROWS_PER_TILE)])


_sc_segment_sum = functools.partial(
    pl.kernel,
    out_type=jax.ShapeDtypeStruct((NC * N_PAD, D), jnp.float32),
    mesh=plsc.VectorSubcoreMesh(core_axis_name="c", subcore_axis_name="s"),
    scratch_types=[
        pltpu.VMEM((SB_C, CHUNK), jnp.int32),
        pltpu.VMEM((SB_C, CHUNK), jnp.int32),
        pltpu.VMEM((CHUNK, D), jnp.float32),
        pltpu.VMEM((CHUNK, D), jnp.float32),
        pltpu.VMEM_SHARED((N_PAD, D), jnp.float32),
        pltpu.SemaphoreType.DMA,
        pltpu.SemaphoreType.DMA,
    ],
)(_sc_segment_sum_body)


BN = 2000


def _mm_relu_body(h_ref, p0_ref, p1_ref, w_ref, b_ref, o_ref):
    s = h_ref[...] + p0_ref[...] + p1_ref[...]
    y = jnp.dot(s, w_ref[...], preferred_element_type=jnp.float32) + b_ref[...]
    o_ref[...] = jnp.maximum(y, 0.0)


def _mm_relu(h, p0, p1, w, b):
    return pl.pallas_call(
        _mm_relu_body,
        out_shape=jax.ShapeDtypeStruct((N, H), jnp.float32),
        grid=(N // BN,),
        in_specs=[
            pl.BlockSpec((BN, D), lambda i: (i, 0)),
            pl.BlockSpec((BN, D), lambda i: (i, 0)),
            pl.BlockSpec((BN, D), lambda i: (i, 0)),
            pl.BlockSpec((D, H), lambda i: (0, 0)),
            pl.BlockSpec((1, H), lambda i: (0, 0)),
        ],
        out_specs=pl.BlockSpec((BN, H), lambda i: (i, 0)),
    )(h, p0, p1, w, b.reshape(1, H))


def _mm2_head_body(h_ref, p0_ref, p1_ref, w_ref, b_ref, wf_ref, bf_ref,
                   o_ref, acc_ref):
    i = pl.program_id(0)
    s = h_ref[...] + p0_ref[...] + p1_ref[...]
    y = jnp.dot(s, w_ref[...], preferred_element_type=jnp.float32) + b_ref[...]
    h2 = jnp.maximum(y, 0.0)
    colsum = jnp.sum(h2, axis=0, keepdims=True)

    @pl.when(i == 0)
    def _():
        acc_ref[...] = colsum

    @pl.when(i > 0)
    def _():
        acc_ref[...] = acc_ref[...] + colsum

    @pl.when(i == pl.num_programs(0) - 1)
    def _():
        mean = acc_ref[...] * (1.0 / N)
        o_ref[...] = (jnp.dot(mean, wf_ref[...],
                              preferred_element_type=jnp.float32) + bf_ref[...])


def _mm2_head(h, p0, p1, w, b, wf, bf):
    return pl.pallas_call(
        _mm2_head_body,
        out_shape=jax.ShapeDtypeStruct((1, C), jnp.float32),
        grid=(N // BN,),
        in_specs=[
            pl.BlockSpec((BN, D), lambda i: (i, 0)),
            pl.BlockSpec((BN, D), lambda i: (i, 0)),
            pl.BlockSpec((BN, D), lambda i: (i, 0)),
            pl.BlockSpec((D, H), lambda i: (0, 0)),
            pl.BlockSpec((1, H), lambda i: (0, 0)),
            pl.BlockSpec((H, C), lambda i: (0, 0)),
            pl.BlockSpec((1, C), lambda i: (0, 0)),
        ],
        out_specs=pl.BlockSpec((1, C), lambda i: (0, 0)),
        scratch_shapes=[pltpu.VMEM((1, H), jnp.float32)],
    )(h, p0, p1, w, b.reshape(1, H), wf, bf.reshape(1, C))


def kernel(x, edge_index, W1, b1, W2, b2, Wf, bf):
    src = edge_index[0].astype(jnp.int32).reshape(NW, SB_N, SB_C, CHUNK)
    dst = edge_index[1].astype(jnp.int32).reshape(NW, SB_N, SB_C, CHUNK)
    zeros = jnp.zeros((N_PAD, D), jnp.float32)

    p = _sc_segment_sum(x, src, dst, zeros)
    h1 = _mm_relu(x, p[:N], p[N_PAD:N_PAD + N], W1, b1)
    p2 = _sc_segment_sum(h1, src, dst, zeros)
    return _mm2_head(h1, p2[:N], p2[N_PAD:N_PAD + N], W2, b2, Wf, bf)
